# baseline (speedup 1.0000x reference)
import functools

import jax
import jax.numpy as jnp
from jax import lax
from jax.experimental import pallas as pl
from jax.experimental.pallas import tpu as pltpu
from jax.experimental.pallas import tpu_sc as plsc

_B, _A, _C, _M = 8, 100000, 80, 32
_BLKA = 2048
_NBLK = 49
_APAD = _BLKA * _NBLK
_NW = 32
_WPI = _NW // _B
_Q = _APAD // _WPI
_CH = 3584
_NCH = _Q // _CH
_NV = _CH // 16
_LN2 = 0.6931471805599453


def _main_kernel(ann_ref, ax0_ref, ay0_ref, ax1_ref, ay1_ref,
                 cls_ref, r0_ref, r1_ref, r2_ref, r3_ref,
                 srow_ref, w_ref, kidx_ref, stats_ref, acc_ref):
    i = pl.program_id(0)

    @pl.when(i == 0)
    def _init():
        acc_ref[...] = jnp.zeros_like(acc_ref)

    ax0 = ax0_ref[...][None, :]
    ay0 = ay0_ref[...][None, :]
    ax1 = ax1_ref[...][None, :]
    ay1 = ay1_ref[...][None, :]
    aw = ax1 - ax0
    ah = ay1 - ay0
    acx = ax0 + 0.5 * aw
    acy = ay0 + 0.5 * ah
    area_a = aw * ah

    ann = ann_ref[...]
    nb = ann.shape[1]

    ib = jnp.full((nb, _BLKA), -1.0, dtype=jnp.float32)
    ub = jnp.ones_like(ib)
    bcx = jnp.zeros_like(ib)
    bcy = jnp.zeros_like(ib)
    bwc = jnp.ones_like(ib)
    bhc = jnp.ones_like(ib)
    blab = jnp.zeros_like(ib)

    for m in range(_M):
        bx0 = ann[0, :, m][:, None]
        by0 = ann[1, :, m][:, None]
        bx1 = ann[2, :, m][:, None]
        by1 = ann[3, :, m][:, None]
        area_b = ann[4, :, m][:, None]
        iw = jnp.minimum(ax1, bx1) - jnp.maximum(ax0, bx0)
        ih = jnp.minimum(ay1, by1) - jnp.maximum(ay0, by0)
        iw = jnp.maximum(iw, 0.0)
        ih = jnp.maximum(ih, 0.0)
        inter = iw * ih
        ua = (area_a + area_b) - inter
        upd = inter * ub > ib * ua
        ib = jnp.where(upd, inter, ib)
        ub = jnp.where(upd, ua, ub)
        bcx = jnp.where(upd, ann[5, :, m][:, None], bcx)
        bcy = jnp.where(upd, ann[6, :, m][:, None], bcy)
        bwc = jnp.where(upd, ann[7, :, m][:, None], bwc)
        bhc = jnp.where(upd, ann[8, :, m][:, None], bhc)
        blab = jnp.where(upd, ann[9, :, m][:, None], blab)

    best = ib / jnp.maximum(ub, 1e-8)
    aid = jax.lax.broadcasted_iota(jnp.int32, (1, _BLKA), 1) + i * _BLKA
    valid = aid < _A
    pos = jnp.logical_and(best >= 0.5, valid)
    neg = jnp.logical_and(best < 0.4, valid)

    c = cls_ref[...]
    srow_ref[...] = jnp.sum((c * c) * jnp.log(1.0 - c), axis=2)
    w_ref[...] = jnp.where(jnp.logical_or(pos, neg), -0.75, 0.0)

    bio = jax.lax.broadcasted_iota(jnp.int32, (nb, 1), 0)
    aid_c = jnp.minimum(aid, _A - 1)
    ki = bio * (_A * _C) + aid_c * _C + blab.astype(jnp.int32)
    kidx_ref[...] = jnp.where(pos, ki + 1, 0)

    raw = 1.0 / jnp.maximum(aw, 1e-6)
    rah = 1.0 / jnp.maximum(ah, 1e-6)
    t0 = (bcx - acx) * raw * 10.0
    t1 = (bcy - acy) * rah * 10.0
    t2 = jnp.log(bwc * raw) * 5.0
    t3 = jnp.log(bhc * rah) * 5.0
    rsum = jnp.zeros_like(ib)
    for t, rref in ((t0, r0_ref), (t1, r1_ref), (t2, r2_ref), (t3, r3_ref)):
        diff = jnp.abs(t - rref[...])
        rsum += jnp.where(diff <= 1.0 / 9.0, 0.5 * 9.0 * diff * diff,
                          diff - 0.5 / 9.0)
    acc_ref[0, :] += jnp.sum(jnp.where(pos, rsum, 0.0), axis=1)
    acc_ref[1, :] += jnp.sum(jnp.where(pos, 1.0, 0.0), axis=1)

    @pl.when(i == _NBLK - 1)
    def _fin():
        stats_ref[...] = acc_ref[...]


def _ln(x):
    bits = lax.bitcast_convert_type(x, jnp.int32)
    ex = lax.shift_right_arithmetic(bits, 23) - 127
    m = lax.bitcast_convert_type(
        (bits & 0x007FFFFF) | 0x3F800000, jnp.float32)
    s = (m - 1.0) / (m + 1.0)
    s2 = s * s
    p = 2.0 * s * (1.0 + s2 * (1.0 / 3.0 + s2 * (
        1.0 / 5.0 + s2 * (1.0 / 7.0 + s2 * (1.0 / 9.0)))))
    return ex.astype(jnp.float32) * _LN2 + p


def _sc_body(cls_hbm, kidx_hbm, out_hbm, kv, gv, ckv, acc_cls, sem):
    wid = lax.axis_index("s") * 2 + lax.axis_index("c")
    b = wid // _WPI
    q = wid % _WPI

    acc_cls[...] = jnp.zeros((16,), jnp.float32)
    zero = jnp.zeros((16,), jnp.float32)

    def chunk_body(t, carry):
        foff = b * _APAD + q * _Q + t * _CH
        pltpu.sync_copy(kidx_hbm.at[pl.ds(foff, _CH)], kv)

        def idx_body(v, c2):
            sl = pl.ds(v * 16, 16)
            gv[sl] = jnp.maximum(kv[sl] - 1, 0)
            return c2

        lax.fori_loop(0, _NV, idx_body, 0)

        gcps = [
            pltpu.async_copy(
                cls_hbm.at[gv.at[pl.ds(j * 128, 128)]],
                ckv.at[pl.ds(j * 128, 128)], sem)
            for j in range(_CH // 128)
        ]
        for cp in gcps:
            cp.wait()

        def corr_body(v, c2):
            sl = pl.ds(v * 16, 16)
            posb = kv[sl] > 0
            ck = jnp.clip(ckv[sl], 1e-6, 1.0 - 1e-6)
            sk = (ck * ck) * _ln(1.0 - ck)
            pos_term = 0.25 * (1.0 - ck) * (1.0 - ck) * (-_ln(ck))
            add = 0.75 * sk + pos_term
            acc_cls[...] = acc_cls[...] + jnp.where(posb, add, zero)
            return c2

        lax.fori_loop(0, _NV, corr_body, 0)
        return carry

    lax.fori_loop(0, _NCH, chunk_body, 0)
    pltpu.sync_copy(acc_cls, out_hbm.at[pl.ds(wid * 16, 16)])


def _wdot_kernel(s_ref, w_ref, out_ref, acc_ref):
    i = pl.program_id(0)

    @pl.when(i == 0)
    def _init():
        acc_ref[...] = jnp.zeros_like(acc_ref)

    wv = w_ref[...]
    acc_ref[0, :] += jnp.sum(
        jnp.where(wv != 0.0, s_ref[...] * wv, 0.0), axis=1)

    @pl.when(i == _NBLK - 1)
    def _fin():
        out_ref[0, :] = acc_ref[0, :]


def kernel(classifications, regressions, anchors, annotations):
    B, A, C = classifications.shape
    M = annotations.shape[1]

    a = anchors[0]
    ax0, ay0, ax1, ay1 = a[:, 0], a[:, 1], a[:, 2], a[:, 3]

    bx0 = annotations[:, :, 0]
    by0 = annotations[:, :, 1]
    bx1 = annotations[:, :, 2]
    by1 = annotations[:, :, 3]
    bw = bx1 - bx0
    bh = by1 - by0
    ann = jnp.stack([
        bx0, by0, bx1, by1,
        bw * bh,
        bx0 + 0.5 * bw,
        by0 + 0.5 * bh,
        jnp.clip(bw, 1.0, None),
        jnp.clip(bh, 1.0, None),
        annotations[:, :, 4],
    ])

    r0 = regressions[:, :, 0]
    r1 = regressions[:, :, 1]
    r2 = regressions[:, :, 2]
    r3 = regressions[:, :, 3]

    srow, w, kidx, stats = pl.pallas_call(
        _main_kernel,
        grid=(_NBLK,),
        in_specs=[
            pl.BlockSpec((10, B, M), lambda i: (0, 0, 0)),
            pl.BlockSpec((_BLKA,), lambda i: (i,)),
            pl.BlockSpec((_BLKA,), lambda i: (i,)),
            pl.BlockSpec((_BLKA,), lambda i: (i,)),
            pl.BlockSpec((_BLKA,), lambda i: (i,)),
            pl.BlockSpec((B, _BLKA, C), lambda i: (0, i, 0)),
            pl.BlockSpec((B, _BLKA), lambda i: (0, i)),
            pl.BlockSpec((B, _BLKA), lambda i: (0, i)),
            pl.BlockSpec((B, _BLKA), lambda i: (0, i)),
            pl.BlockSpec((B, _BLKA), lambda i: (0, i)),
        ],
        out_specs=[
            pl.BlockSpec((B, _BLKA), lambda i: (0, i)),
            pl.BlockSpec((B, _BLKA), lambda i: (0, i)),
            pl.BlockSpec((B, _BLKA), lambda i: (0, i)),
            pl.BlockSpec((2, B), lambda i: (0, 0)),
        ],
        out_shape=[
            jax.ShapeDtypeStruct((B, _APAD), jnp.float32),
            jax.ShapeDtypeStruct((B, _APAD), jnp.float32),
            jax.ShapeDtypeStruct((B, _APAD), jnp.int32),
            jax.ShapeDtypeStruct((2, B), jnp.float32),
        ],
        scratch_shapes=[pltpu.VMEM((2, B), jnp.float32)],
    )(ann, ax0, ay0, ax1, ay1, classifications, r0, r1, r2, r3)

    cls_flat = classifications.reshape(-1)
    sc = functools.partial(
        pl.kernel,
        out_type=jax.ShapeDtypeStruct((_NW * 16,), jnp.float32),
        mesh=plsc.VectorSubcoreMesh(core_axis_name="c",
                                    subcore_axis_name="s"),
        scratch_types=[
            pltpu.VMEM((_CH,), jnp.int32),
            pltpu.VMEM((_CH,), jnp.int32),
            pltpu.VMEM((_CH,), jnp.float32),
            pltpu.VMEM((16,), jnp.float32),
            pltpu.SemaphoreType.DMA,
        ],
    )(_sc_body)
    parts = sc(cls_flat, kidx.reshape(-1))

    wdot = pl.pallas_call(
        _wdot_kernel,
        grid=(_NBLK,),
        in_specs=[pl.BlockSpec((B, _BLKA), lambda i: (0, i)),
                  pl.BlockSpec((B, _BLKA), lambda i: (0, i))],
        out_specs=pl.BlockSpec((1, B), lambda i: (0, 0)),
        out_shape=jax.ShapeDtypeStruct((1, B), jnp.float32),
        scratch_shapes=[pltpu.VMEM((1, B), jnp.float32)],
    )(srow, w)[0]

    add_sum = jnp.sum(parts.reshape(B, _WPI * 16), axis=1)
    cls_sum = wdot + add_sum
    rgs_sum = stats[0]
    npos = stats[1]
    cls_out = cls_sum / jnp.maximum(npos, 1.0)
    rgs_out = jnp.where(npos > 0.0,
                        rgs_sum / jnp.maximum(npos * 4.0, 1.0), 0.0)
    return jnp.stack([cls_out, rgs_out])


# speedup vs baseline: 3.9762x; 3.9762x over previous
import functools

import jax
import jax.numpy as jnp
from jax import lax
from jax.experimental import pallas as pl
from jax.experimental.pallas import tpu as pltpu
from jax.experimental.pallas import tpu_sc as plsc

_B, _A, _C, _M = 8, 100000, 80, 32
_BLKA = 2048
_NBLK = 49
_APAD = _BLKA * _NBLK
_NW = 32
_WPI = _NW // _B
_Q = _APAD // _WPI
_CH = 3584
_NCH = _Q // _CH
_NV = _CH // 16
_LN2 = 0.6931471805599453


def _srow_kernel(cls_ref, out_ref):
    c = cls_ref[...]
    out_ref[...] = jnp.sum((c * c) * jnp.log(1.0 - c), axis=2)


def _ln(x):
    bits = lax.bitcast_convert_type(x, jnp.int32)
    ex = lax.shift_right_arithmetic(bits, 23) - 127
    m = lax.bitcast_convert_type(
        (bits & 0x007FFFFF) | 0x3F800000, jnp.float32)
    s = (m - 1.0) / (m + 1.0)
    s2 = s * s
    p = 2.0 * s * (1.0 + s2 * (1.0 / 3.0 + s2 * (
        1.0 / 5.0 + s2 * (1.0 / 7.0 + s2 * (1.0 / 9.0)))))
    return ex.astype(jnp.float32) * _LN2 + p


def _wdot_kernel(s_ref, w_ref, out_ref, acc_ref):
    i = pl.program_id(0)

    @pl.when(i == 0)
    def _init():
        acc_ref[...] = jnp.zeros_like(acc_ref)

    wv = w_ref[...]
    acc_ref[0, :] += jnp.sum(
        jnp.where(wv != 0.0, s_ref[...] * wv, 0.0), axis=1)

    @pl.when(i == _NBLK - 1)
    def _fin():
        out_ref[0, :] = acc_ref[0, :]


def _sc_body(cls_hbm, ax0_hbm, ay0_hbm, ax1_hbm, ay1_hbm,
             r0_hbm, r1_hbm, r2_hbm, r3_hbm, ann_hbm, out_hbm, w_hbm,
             annv, btab, ax0v, ay0v, ax1v, ay1v, wv,
             r0v, r1v, r2v, r3v, kidxv, ckv, posv, negv,
             acc_cls, acc_rgs, acc_np, sem):
    wid = lax.axis_index("s") * 2 + lax.axis_index("c")
    b = wid // _WPI
    q = wid % _WPI

    pltpu.sync_copy(ann_hbm.at[pl.ds(b * 10 * _M, 10 * _M)], annv)

    for j in range(5):
        for m in range(_M):
            val = annv[pl.ds(j * _M + m, 16)][0]
            btab[pl.ds((j * _M + m) * 16, 16)] = jnp.full(
                (16,), val, jnp.float32)

    fld = []
    for j in range(5, 10):
        fld.append((annv[pl.ds(j * _M, 16)], annv[pl.ds(j * _M + 16, 16)]))

    acc_cls[...] = jnp.zeros((16,), jnp.float32)
    acc_rgs[...] = jnp.zeros((16,), jnp.float32)
    acc_np[...] = jnp.zeros((16,), jnp.float32)

    lanes = lax.broadcasted_iota(jnp.int32, (16,), 0)
    zero = jnp.zeros((16,), jnp.float32)

    def chunk_body(t, carry):
        base = q * _Q + t * _CH
        foff = b * _APAD + base
        cps = [
            pltpu.async_copy(ax0_hbm.at[pl.ds(base, _CH)], ax0v, sem),
            pltpu.async_copy(ay0_hbm.at[pl.ds(base, _CH)], ay0v, sem),
            pltpu.async_copy(ax1_hbm.at[pl.ds(base, _CH)], ax1v, sem),
            pltpu.async_copy(ay1_hbm.at[pl.ds(base, _CH)], ay1v, sem),
            pltpu.async_copy(r0_hbm.at[pl.ds(foff, _CH)], r0v, sem),
            pltpu.async_copy(r1_hbm.at[pl.ds(foff, _CH)], r1v, sem),
            pltpu.async_copy(r2_hbm.at[pl.ds(foff, _CH)], r2v, sem),
            pltpu.async_copy(r3_hbm.at[pl.ds(foff, _CH)], r3v, sem),
        ]
        for cp in cps:
            cp.wait()

        def match_body(v, c2):
            sl = pl.ds(v * 16, 16)
            ax0 = ax0v[sl]
            ay0 = ay0v[sl]
            ax1 = ax1v[sl]
            ay1 = ay1v[sl]
            aw = ax1 - ax0
            ah = ay1 - ay0
            area_a = aw * ah
            ib = jnp.full((16,), -1.0, jnp.float32)
            ub = jnp.ones((16,), jnp.float32)
            mb = jnp.zeros((16,), jnp.int32)
            for m in range(_M):
                bx0 = btab[pl.ds((0 * _M + m) * 16, 16)]
                by0 = btab[pl.ds((1 * _M + m) * 16, 16)]
                bx1 = btab[pl.ds((2 * _M + m) * 16, 16)]
                by1 = btab[pl.ds((3 * _M + m) * 16, 16)]
                areab = btab[pl.ds((4 * _M + m) * 16, 16)]
                iw = jnp.minimum(ax1, bx1) - jnp.maximum(ax0, bx0)
                ih = jnp.minimum(ay1, by1) - jnp.maximum(ay0, by0)
                iw = jnp.maximum(iw, 0.0)
                ih = jnp.maximum(ih, 0.0)
                inter = iw * ih
                ua = (area_a + areab) - inter
                upd = inter * ub > ib * ua
                ib = jnp.where(upd, inter, ib)
                ub = jnp.where(upd, ua, ub)
                mb = jnp.where(upd, jnp.int32(m), mb)
            best = ib / jnp.maximum(ub, 1e-8)
            g = base + v * 16 + lanes
            validm = g < _A
            posb = jnp.logical_and(best >= 0.5, validm)
            negb = jnp.logical_and(best < 0.4, validm)

            mlo = jnp.minimum(mb, 15)
            mhi = jnp.maximum(mb - 16, 0)
            lowh = mb < 16

            def dyng(v, idx):
                return lax.gather(
                    v, idx[:, None],
                    lax.GatherDimensionNumbers(
                        offset_dims=(), collapsed_slice_dims=(0,),
                        start_index_map=(0,)),
                    (1,), mode=lax.GatherScatterMode.PROMISE_IN_BOUNDS)

            def pick(pair):
                return jnp.where(lowh, dyng(pair[0], mlo),
                                 dyng(pair[1], mhi))

            cxg = pick(fld[0])
            cyg = pick(fld[1])
            wcg = pick(fld[2])
            hcg = pick(fld[3])
            labg = pick(fld[4])

            acx = ax0 + 0.5 * aw
            acy = ay0 + 0.5 * ah
            t0 = ((cxg - acx) / aw) * 10.0
            t1 = ((cyg - acy) / ah) * 10.0
            t2 = _ln(wcg / aw) * 5.0
            t3 = _ln(hcg / ah) * 5.0
            rsum = zero
            for tt, rv in ((t0, r0v), (t1, r1v), (t2, r2v), (t3, r3v)):
                diff = jnp.abs(tt - rv[sl])
                rsum = rsum + jnp.where(diff <= 1.0 / 9.0,
                                        4.5 * diff * diff,
                                        diff - 0.5 / 9.0)
            acc_rgs[...] = acc_rgs[...] + jnp.where(posb, rsum, zero)
            acc_np[...] = acc_np[...] + jnp.where(
                posb, jnp.ones((16,), jnp.float32), zero)

            posv[sl] = jnp.where(posb, jnp.ones((16,), jnp.float32), zero)
            wv[sl] = jnp.where(jnp.logical_or(posb, negb),
                               jnp.full((16,), -0.75, jnp.float32), zero)
            gi = jnp.minimum(g, _A - 1)
            kidxv[sl] = (b * _A + gi) * _C + labg.astype(jnp.int32)
            return c2

        lax.fori_loop(0, _NV, match_body, 0)

        gcps = [
            pltpu.async_copy(
                cls_hbm.at[kidxv.at[pl.ds(j * 128, 128)]],
                ckv.at[pl.ds(j * 128, 128)], sem)
            for j in range(_CH // 128)
        ]
        for cp in gcps:
            cp.wait()

        def corr_body(v, c2):
            sl = pl.ds(v * 16, 16)
            ck = jnp.clip(ckv[sl], 1e-6, 1.0 - 1e-6)
            posf = posv[sl]
            ln1m = _ln(1.0 - ck)
            lnck = _ln(ck)
            sk = (ck * ck) * ln1m
            pos_term = 0.25 * (1.0 - ck) * (1.0 - ck) * (-lnck)
            add = 0.75 * sk + pos_term
            acc_cls[...] = acc_cls[...] + jnp.where(posf > 0.5, add, zero)
            return c2

        lax.fori_loop(0, _NV, corr_body, 0)
        pltpu.sync_copy(wv, w_hbm.at[pl.ds(foff, _CH)])
        return carry

    lax.fori_loop(0, _NCH, chunk_body, 0)

    pltpu.sync_copy(acc_cls, out_hbm.at[pl.ds(wid * 48, 16)])
    pltpu.sync_copy(acc_rgs, out_hbm.at[pl.ds(wid * 48 + 16, 16)])
    pltpu.sync_copy(acc_np, out_hbm.at[pl.ds(wid * 48 + 32, 16)])


def kernel(classifications, regressions, anchors, annotations):
    B, A, C = classifications.shape
    M = annotations.shape[1]

    srow = pl.pallas_call(
        _srow_kernel,
        grid=(_NBLK,),
        in_specs=[pl.BlockSpec((B, _BLKA, C), lambda i: (0, i, 0))],
        out_specs=pl.BlockSpec((B, _BLKA), lambda i: (0, i)),
        out_shape=jax.ShapeDtypeStruct((B, _APAD), jnp.float32),
    )(classifications)

    a = anchors[0]
    pad = _APAD - A
    ax0 = jnp.pad(a[:, 0], (0, pad))
    ay0 = jnp.pad(a[:, 1], (0, pad))
    ax1 = jnp.pad(a[:, 2], (0, pad))
    ay1 = jnp.pad(a[:, 3], (0, pad))
    r0 = jnp.pad(regressions[:, :, 0], ((0, 0), (0, pad)))
    r1 = jnp.pad(regressions[:, :, 1], ((0, 0), (0, pad)))
    r2 = jnp.pad(regressions[:, :, 2], ((0, 0), (0, pad)))
    r3 = jnp.pad(regressions[:, :, 3], ((0, 0), (0, pad)))

    bx0 = annotations[:, :, 0]
    by0 = annotations[:, :, 1]
    bx1 = annotations[:, :, 2]
    by1 = annotations[:, :, 3]
    bw = bx1 - bx0
    bh = by1 - by0
    ann = jnp.stack([
        bx0, by0, bx1, by1,
        bw * bh,
        bx0 + 0.5 * bw,
        by0 + 0.5 * bh,
        jnp.clip(bw, 1.0, None),
        jnp.clip(bh, 1.0, None),
        annotations[:, :, 4],
    ], axis=1).reshape(B * 10 * M)

    cls_flat = classifications.reshape(-1)

    sc = functools.partial(
        pl.kernel,
        out_type=(jax.ShapeDtypeStruct((_NW * 3 * 16,), jnp.float32),
                  jax.ShapeDtypeStruct((_B * _APAD,), jnp.float32)),
        mesh=plsc.VectorSubcoreMesh(core_axis_name="c",
                                    subcore_axis_name="s"),
        scratch_types=[
            pltpu.VMEM((10 * M,), jnp.float32),
            pltpu.VMEM((5 * M * 16,), jnp.float32),
            pltpu.VMEM((_CH,), jnp.float32),
            pltpu.VMEM((_CH,), jnp.float32),
            pltpu.VMEM((_CH,), jnp.float32),
            pltpu.VMEM((_CH,), jnp.float32),
            pltpu.VMEM((_CH,), jnp.float32),
            pltpu.VMEM((_CH,), jnp.float32),
            pltpu.VMEM((_CH,), jnp.float32),
            pltpu.VMEM((_CH,), jnp.float32),
            pltpu.VMEM((_CH,), jnp.float32),
            pltpu.VMEM((_CH,), jnp.int32),
            pltpu.VMEM((_CH,), jnp.float32),
            pltpu.VMEM((_CH,), jnp.float32),
            pltpu.VMEM((_CH,), jnp.float32),
            pltpu.VMEM((16,), jnp.float32),
            pltpu.VMEM((16,), jnp.float32),
            pltpu.VMEM((16,), jnp.float32),
            pltpu.SemaphoreType.DMA,
        ],
    )(_sc_body)
    parts, w = sc(cls_flat, ax0, ay0, ax1, ay1,
                  r0.reshape(-1), r1.reshape(-1), r2.reshape(-1),
                  r3.reshape(-1), ann)

    wdot = pl.pallas_call(
        _wdot_kernel,
        grid=(_NBLK,),
        in_specs=[pl.BlockSpec((B, _BLKA), lambda i: (0, i)),
                  pl.BlockSpec((B, _BLKA), lambda i: (0, i))],
        out_specs=pl.BlockSpec((1, B), lambda i: (0, 0)),
        out_shape=jax.ShapeDtypeStruct((1, B), jnp.float32),
        scratch_shapes=[pltpu.VMEM((1, B), jnp.float32)],
    )(srow, w.reshape(B, _APAD))[0]

    parts = parts.reshape(B, _WPI, 3, 16)
    cls_sum = wdot + jnp.sum(parts[:, :, 0, :], axis=(1, 2))
    rgs_sum = jnp.sum(parts[:, :, 1, :], axis=(1, 2))
    npos = jnp.sum(parts[:, :, 2, :], axis=(1, 2))
    cls_out = cls_sum / jnp.maximum(npos, 1.0)
    rgs_out = jnp.where(npos > 0.0,
                        rgs_sum / jnp.maximum(npos * 4.0, 1.0), 0.0)
    return jnp.stack([cls_out, rgs_out])
